# baseline (speedup 1.0000x reference)
import jax
import jax.numpy as jnp
from jax.experimental import pallas as pl
from jax.experimental.pallas import tpu as pltpu

_R = 128
_B = 32768
_C = 256
_N = 32768


def _scan_body(x_ref, tri_ref, o_ref, carry_ref):
    i = pl.program_id(0)

    @pl.when(i == 0)
    def _():
        carry_ref[...] = jnp.zeros_like(carry_ref)

    tri = tri_ref[...]
    off = carry_ref[:, 0:1]
    for c in range(_B // _C):
        blk = x_ref[:, c * _C:(c + 1) * _C]
        hi = blk.astype(jnp.bfloat16)
        lo = (blk - hi.astype(jnp.float32)).astype(jnp.bfloat16)
        cs = jax.lax.dot(hi, tri, preferred_element_type=jnp.float32)
        cs = cs + jax.lax.dot(lo, tri, preferred_element_type=jnp.float32)
        o_ref[:, c * _C:(c + 1) * _C] = cs + off
        off = off + cs[:, _C - 1:_C]
    carry_ref[...] = jnp.broadcast_to(off, carry_ref.shape)


def kernel(x):
    tri = jnp.triu(jnp.ones((_C, _C), dtype=jnp.bfloat16))
    grid = (_N // _B,)
    return pl.pallas_call(
        _scan_body,
        grid=grid,
        in_specs=[
            pl.BlockSpec((_R, _B), lambda i: (0, i)),
            pl.BlockSpec((_C, _C), lambda i: (0, 0)),
        ],
        out_specs=pl.BlockSpec((_R, _B), lambda i: (0, i)),
        out_shape=jax.ShapeDtypeStruct((_R, _N), jnp.float32),
        scratch_shapes=[pltpu.VMEM((_R, 128), jnp.float32)],
        compiler_params=pltpu.CompilerParams(
            dimension_semantics=("arbitrary",),
        ),
    )(x, tri)


# speedup vs baseline: 1.0188x; 1.0188x over previous
import jax
import jax.numpy as jnp
from jax.experimental import pallas as pl
from jax.experimental.pallas import tpu as pltpu

_R = 128
_B = 16384
_C = 256
_N = 32768
_NC = _B // _C


def _scan_body(x_ref, tri_ref, texc_ref, o_ref, carry_ref, t_ref):
    i = pl.program_id(0)

    @pl.when(i == 0)
    def _():
        carry_ref[...] = jnp.zeros_like(carry_ref)

    tri = tri_ref[...]
    for c in range(_NC):
        blk = x_ref[:, c * _C:(c + 1) * _C]
        hi = blk.astype(jnp.bfloat16)
        lo = (blk - hi.astype(jnp.float32)).astype(jnp.bfloat16)
        cs = jax.lax.dot(hi, tri, preferred_element_type=jnp.float32)
        cs = cs + jax.lax.dot(lo, tri, preferred_element_type=jnp.float32)
        o_ref[:, c * _C:(c + 1) * _C] = cs
        t_ref[:, c:c + 1] = cs[:, _C - 1:_C]

    off = jax.lax.dot(t_ref[...], texc_ref[...],
                      precision=jax.lax.Precision.HIGHEST)
    off = off + carry_ref[:, 0:1]

    for c in range(_NC):
        o_ref[:, c * _C:(c + 1) * _C] += off[:, c:c + 1]

    carry_ref[...] = jnp.broadcast_to(
        off[:, _NC - 1:_NC] + t_ref[:, _NC - 1:_NC], carry_ref.shape)


def kernel(x):
    tri = jnp.triu(jnp.ones((_C, _C), dtype=jnp.bfloat16))
    texc = jnp.triu(jnp.ones((_NC, _NC), dtype=jnp.float32), k=1)
    grid = (_N // _B,)
    return pl.pallas_call(
        _scan_body,
        grid=grid,
        in_specs=[
            pl.BlockSpec((_R, _B), lambda i: (0, i)),
            pl.BlockSpec((_C, _C), lambda i: (0, 0)),
            pl.BlockSpec((_NC, _NC), lambda i: (0, 0)),
        ],
        out_specs=pl.BlockSpec((_R, _B), lambda i: (0, i)),
        out_shape=jax.ShapeDtypeStruct((_R, _N), jnp.float32),
        scratch_shapes=[
            pltpu.VMEM((_R, 128), jnp.float32),
            pltpu.VMEM((_R, _NC), jnp.float32),
        ],
        compiler_params=pltpu.CompilerParams(
            dimension_semantics=("arbitrary",),
        ),
    )(x, tri, texc)
